# Initial kernel scaffold; baseline (speedup 1.0000x reference)
import jax
import jax.numpy as jnp
from jax.experimental import pallas as pl


def kernel(x, ei, ew, b, W1, b1, W2, b2, M1, mb1, M2, mb2):
    raise NotImplementedError("write your pallas kernel here")


# speedup vs baseline: 8.7660x; 8.7660x over previous
import functools

import jax
import jax.numpy as jnp
from jax import lax
from jax.experimental import pallas as pl
from jax.experimental.pallas import tpu as pltpu
from jax.experimental.pallas import tpu_sc as plsc

N = 10000
E = 320000
D = 128
H = 64
B = 32

NPAD = 10240
NC = 2
NS = 16
NW = NC * NS
K1 = 80
C1 = E // (NW * K1)
RPT = NPAD // NS


def _sc_mesh():
    return plsc.VectorSubcoreMesh(core_axis_name="c", subcore_axis_name="s")


_SC_PARAMS = pltpu.CompilerParams(use_tc_tiling_on_sc=False)


def _deg_kernel(dst3, ew3):
    @functools.partial(
        pl.kernel,
        out_type=jax.ShapeDtypeStruct((NC, NPAD), jnp.float32),
        mesh=_sc_mesh(),
        compiler_params=_SC_PARAMS,
        scratch_types=[
            pltpu.VMEM((K1,), jnp.int32),
            pltpu.VMEM((K1,), jnp.float32),
            pltpu.VMEM((RPT,), jnp.float32),
            pltpu.VMEM_SHARED((NPAD,), jnp.float32),
        ],
    )
    def k(dst_h, ew_h, out_h, dstv, ewv, zbuf, acc):
        cid = lax.axis_index("c")
        sid = lax.axis_index("s")
        wid = cid * NS + sid

        for i in range(RPT // 16):
            zbuf[pl.ds(i * 16, 16)] = jnp.zeros((16,), jnp.float32)
        pltpu.sync_copy(zbuf, acc.at[pl.ds(sid * RPT, RPT)])
        plsc.subcore_barrier()

        def body(c, carry):
            pltpu.sync_copy(dst_h.at[wid, c], dstv)
            pltpu.sync_copy(ew_h.at[wid, c], ewv)
            pltpu.sync_copy(ewv, acc.at[dstv], add=True)
            return carry

        lax.fori_loop(0, C1, body, 0)
        plsc.subcore_barrier()
        pltpu.sync_copy(acc.at[pl.ds(sid * RPT, RPT)],
                        out_h.at[cid, pl.ds(sid * RPT, RPT)])

    return k(dst3, ew3)


def _agg_kernel(g, src3, dst3, ew3):
    @functools.partial(
        pl.kernel,
        out_type=jax.ShapeDtypeStruct((NC, NPAD, H), jnp.float32),
        mesh=_sc_mesh(),
        compiler_params=_SC_PARAMS,
        scratch_types=[
            pltpu.VMEM((K1,), jnp.int32),
            pltpu.VMEM((K1,), jnp.int32),
            pltpu.VMEM((K1,), jnp.float32),
            pltpu.VMEM((K1, H), jnp.float32),
            pltpu.VMEM((64, H), jnp.float32),
            pltpu.VMEM_SHARED((NPAD, H), jnp.float32),
        ],
    )
    def k(g_h, src_h, dst_h, ew_h, out_h, srcv, dstv, ewv, rows, zbuf, acc):
        cid = lax.axis_index("c")
        sid = lax.axis_index("s")
        wid = cid * NS + sid

        def zfill(i, carry):
            for j in range(H // 16):
                zbuf[i, pl.ds(j * 16, 16)] = jnp.zeros((16,), jnp.float32)
            return carry

        lax.fori_loop(0, 64, zfill, 0)

        def zcopy(i, carry):
            pltpu.sync_copy(zbuf, acc.at[pl.ds(sid * RPT + i * 64, 64)])
            return carry

        lax.fori_loop(0, RPT // 64, zcopy, 0)
        plsc.subcore_barrier()

        def body(c, carry):
            pltpu.sync_copy(src_h.at[wid, c], srcv)
            pltpu.sync_copy(dst_h.at[wid, c], dstv)
            pltpu.sync_copy(ew_h.at[wid, c], ewv)
            pltpu.sync_copy(g_h.at[srcv], rows)

            def scale(eg, c2):
                ew16 = ewv[pl.ds(eg * 16, 16)]
                for j in range(16):
                    s = ew16[j]
                    e = eg * 16 + j
                    for f in range(H // 16):
                        sl = pl.ds(f * 16, 16)
                        rows[e, sl] = rows[e, sl] * s
                return c2

            lax.fori_loop(0, K1 // 16, scale, 0)
            pltpu.sync_copy(rows, acc.at[dstv], add=True)
            return carry

        lax.fori_loop(0, C1, body, 0)
        plsc.subcore_barrier()
        pltpu.sync_copy(acc.at[pl.ds(sid * RPT, RPT)],
                        out_h.at[cid, pl.ds(sid * RPT, RPT)])

    return k(g, src3, dst3, ew3)


def _tc1(xp, W1, degp):
    def body(x_ref, w_ref, degp_ref, g_ref, dis_ref):
        deg = degp_ref[0] + degp_ref[1] + 1.0
        dis = lax.rsqrt(deg)
        h = jnp.dot(x_ref[...], w_ref[...], preferred_element_type=jnp.float32)
        g_ref[...] = h * dis
        dis_ref[...] = dis

    return pl.pallas_call(
        body,
        out_shape=(
            jax.ShapeDtypeStruct((NPAD, H), jnp.float32),
            jax.ShapeDtypeStruct((NPAD, 1), jnp.float32),
        ),
    )(xp, W1, degp)


def _tc2(aggp, g1, dis, W2, b1row):
    def body(aggp_ref, g1_ref, dis_ref, w_ref, b_ref, g2_ref):
        dis = dis_ref[...]
        h = jnp.maximum(
            (aggp_ref[0] + aggp_ref[1] + g1_ref[...]) * dis + b_ref[...], 0.0)
        t = jnp.dot(h, w_ref[...], preferred_element_type=jnp.float32)
        g2_ref[...] = t * dis

    return pl.pallas_call(
        body,
        out_shape=jax.ShapeDtypeStruct((NPAD, H), jnp.float32),
    )(aggp, g1, dis, W2, b1row)


def _tc3(aggp, g2, dis, b2row, brow, M1, mb1row, M2p, mb2row):
    def body(aggp_ref, g2_ref, dis_ref, b2_ref, brow_ref, m1_ref, mb1_ref,
             m2_ref, mb2_ref, out_ref):
        dis = dis_ref[...]
        h2 = jnp.maximum(
            (aggp_ref[0] + aggp_ref[1] + g2_ref[...]) * dis + b2_ref[...], 0.0)
        gids = lax.broadcasted_iota(jnp.int32, (B, NPAD), 0)
        oh = (brow_ref[...] == gids).astype(jnp.float32)
        ssum = jnp.dot(oh, h2, preferred_element_type=jnp.float32)
        cnt = jnp.dot(oh, jnp.ones((NPAD, 1), jnp.float32),
                      preferred_element_type=jnp.float32)
        pooled = ssum / jnp.maximum(cnt, 1.0)
        z = jnp.maximum(
            jnp.dot(pooled, m1_ref[...], preferred_element_type=jnp.float32)
            + mb1_ref[...], 0.0)
        out_ref[...] = (
            jnp.dot(z, m2_ref[...], preferred_element_type=jnp.float32)
            + mb2_ref[...])

    return pl.pallas_call(
        body,
        out_shape=jax.ShapeDtypeStruct((B, 128), jnp.float32),
    )(aggp, g2, dis, b2row, brow, M1, mb1row, M2p, mb2row)


@jax.jit
def kernel(x, ei, ew, b, W1, b1, W2, b2, M1, mb1, M2, mb2):
    xp = jnp.pad(x, ((0, NPAD - N), (0, 0)))
    src3 = ei[0].reshape(NW, C1, K1)
    dst3 = ei[1].reshape(NW, C1, K1)
    ew3 = ew.reshape(NW, C1, K1)
    brow = jnp.pad(b, (0, NPAD - N), constant_values=B)[None, :]
    b1row = b1[None, :]
    b2row = b2[None, :]
    mb1row = mb1[None, :]
    M2p = jnp.pad(M2, ((0, 0), (0, 128 - M2.shape[1])))
    mb2row = jnp.pad(mb2, (0, 128 - mb2.shape[0]))[None, :]

    degp = _deg_kernel(dst3, ew3)
    degp3 = degp[:, :, None]

    g1, dis = _tc1(xp, W1, degp3)
    agg1 = _agg_kernel(g1, src3, dst3, ew3)
    g2 = _tc2(agg1, g1, dis, W2, b1row)
    agg2 = _agg_kernel(g2, src3, dst3, ew3)
    out_full = _tc3(agg2, g2, dis, b2row, brow, M1, mb1row, M2p, mb2row)
    return out_full[:, :1]


# speedup vs baseline: 15.5861x; 1.7780x over previous
import functools

import jax
import jax.numpy as jnp
from jax import lax
from jax.experimental import pallas as pl
from jax.experimental.pallas import tpu as pltpu
from jax.experimental.pallas import tpu_sc as plsc

N = 10000
E = 320000
D = 128
H = 64
B = 32

NPAD = 10240
NC = 2
NS = 16
NW = NC * NS
K1 = 128
C1 = 81
EPW = C1 * K1
EPAD = NW * EPW
NBUF = 3
RPT = NPAD // NS
FG = H // 16


def _sc_mesh():
    return plsc.VectorSubcoreMesh(core_axis_name="c", subcore_axis_name="s")


_SC_PARAMS = pltpu.CompilerParams(use_tc_tiling_on_sc=False,
                                  needs_layout_passes=False)


def _deg_kernel(meta):
    @functools.partial(
        pl.kernel,
        out_type=jax.ShapeDtypeStruct((NC, NPAD), jnp.float32),
        mesh=_sc_mesh(),
        compiler_params=_SC_PARAMS,
        scratch_types=(
            [pltpu.VMEM((3, K1), jnp.int32)] * NBUF
            + [pltpu.VMEM((K1,), jnp.float32)] * NBUF
            + [pltpu.VMEM((RPT,), jnp.float32),
               pltpu.VMEM_SHARED((NPAD,), jnp.float32)]
            + [pltpu.SemaphoreType.DMA] * (2 * NBUF)
        ),
    )
    def k(meta_h, out_h, ed0, ed1, ed2, ef0, ef1, ef2, zbuf, acc,
          sm0, sm1, sm2, ss0, ss1, ss2):
        ed = [ed0, ed1, ed2]
        ef = [ef0, ef1, ef2]
        sm = [sm0, sm1, sm2]
        ss = [ss0, ss1, ss2]
        cid = lax.axis_index("c")
        sid = lax.axis_index("s")
        wid = cid * NS + sid

        for i in range(RPT // 16):
            zbuf[pl.ds(i * 16, 16)] = jnp.zeros((16,), jnp.float32)
        pltpu.sync_copy(zbuf, acc.at[pl.ds(sid * RPT, RPT)])
        plsc.subcore_barrier()

        pltpu.async_copy(meta_h.at[wid, 0], ed[0], sm[0])
        pltpu.async_copy(meta_h.at[wid, 1], ed[1], sm[1])

        def body(gi, carry):
            for b in range(NBUF):
                c = gi * NBUF + b
                s, s1, s2 = b, (b + 1) % NBUF, (b + 2) % NBUF
                pltpu.make_async_copy(meta_h.at[wid, c], ed[s], sm[s]).wait()
                for gg in range(K1 // 16):
                    sl = pl.ds(gg * 16, 16)
                    ef[s][sl] = plsc.bitcast(ed[s][2, sl], jnp.float32)

                sdesc = pltpu.async_copy(ef[s], acc.at[ed[s].at[1]], ss[s],
                                         add=True)

                @pl.when(c + 2 < C1)
                def _():
                    pltpu.async_copy(meta_h.at[wid, c + 2], ed[s2], sm[s2])

                sdesc.wait()
            return carry

        lax.fori_loop(0, C1 // NBUF, body, 0)
        plsc.subcore_barrier()
        pltpu.sync_copy(acc.at[pl.ds(sid * RPT, RPT)],
                        out_h.at[cid, pl.ds(sid * RPT, RPT)])

    return k(meta)


def _agg_kernel(g, meta):
    @functools.partial(
        pl.kernel,
        out_type=jax.ShapeDtypeStruct((NC, NPAD, H), jnp.float32),
        mesh=_sc_mesh(),
        compiler_params=_SC_PARAMS,
        scratch_types=(
            [pltpu.VMEM((3, K1), jnp.int32)] * NBUF
            + [pltpu.VMEM((K1, H), jnp.float32)] * NBUF
            + [pltpu.VMEM((64, H), jnp.float32),
               pltpu.VMEM_SHARED((NPAD, H), jnp.float32)]
            + [pltpu.SemaphoreType.DMA] * (3 * NBUF)
        ),
    )
    def k(g_h, meta_h, out_h, ed0, ed1, ed2, r0, r1, r2, zbuf, acc,
          sm0, sm1, sm2, sg0, sg1, sg2, ss0, ss1, ss2):
        ed = [ed0, ed1, ed2]
        rows = [r0, r1, r2]
        sm = [sm0, sm1, sm2]
        sg = [sg0, sg1, sg2]
        ss = [ss0, ss1, ss2]
        cid = lax.axis_index("c")
        sid = lax.axis_index("s")
        wid = cid * NS + sid

        def zfill(i, carry):
            for j in range(FG):
                zbuf[i, pl.ds(j * 16, 16)] = jnp.zeros((16,), jnp.float32)
            return carry

        lax.fori_loop(0, 64, zfill, 0)

        def zcopy(i, carry):
            pltpu.sync_copy(zbuf, acc.at[pl.ds(sid * RPT + i * 64, 64)])
            return carry

        lax.fori_loop(0, RPT // 64, zcopy, 0)
        plsc.subcore_barrier()

        pltpu.async_copy(meta_h.at[wid, 0], ed[0], sm[0])
        pltpu.async_copy(meta_h.at[wid, 1], ed[1], sm[1])
        pltpu.make_async_copy(meta_h.at[wid, 0], ed[0], sm[0]).wait()
        pltpu.async_copy(g_h.at[ed[0].at[0]], rows[0], sg[0])

        def body(gi, carry):
            for b in range(NBUF):
                c = gi * NBUF + b
                s, s1, s2 = b, (b + 1) % NBUF, (b + 2) % NBUF
                pltpu.make_async_copy(
                    g_h.at[ed[s].at[0]], rows[s], sg[s]).wait()
                rs = rows[s]

                def scale(gg, c2):
                    ew16 = plsc.bitcast(ed[s][2, pl.ds(gg * 16, 16)],
                                        jnp.float32)
                    for j in range(16):
                        e = gg * 16 + j
                        w = ew16[j]
                        for f in range(FG):
                            sl = pl.ds(f * 16, 16)
                            rs[e, sl] = rs[e, sl] * w
                    return c2

                lax.fori_loop(0, K1 // 16, scale, 0)

                sdesc = pltpu.async_copy(rows[s], acc.at[ed[s].at[1]], ss[s],
                                         add=True)

                @pl.when(c + 2 < C1)
                def _():
                    pltpu.async_copy(meta_h.at[wid, c + 2], ed[s2], sm[s2])

                @pl.when(c + 1 < C1)
                def _():
                    pltpu.make_async_copy(
                        meta_h.at[wid, c + 1], ed[s1], sm[s1]).wait()
                    pltpu.async_copy(g_h.at[ed[s1].at[0]], rows[s1], sg[s1])

                sdesc.wait()
            return carry

        lax.fori_loop(0, C1 // NBUF, body, 0)
        plsc.subcore_barrier()
        pltpu.sync_copy(acc.at[pl.ds(sid * RPT, RPT)],
                        out_h.at[cid, pl.ds(sid * RPT, RPT)])

    return k(g, meta)


def _tc1(xp, W1, degp):
    def body(x_ref, w_ref, degp_ref, g_ref, dis_ref):
        deg = degp_ref[0] + degp_ref[1] + 1.0
        dis = lax.rsqrt(deg)
        h = jnp.dot(x_ref[...], w_ref[...], preferred_element_type=jnp.float32)
        g_ref[...] = h * dis
        dis_ref[...] = dis

    return pl.pallas_call(
        body,
        out_shape=(
            jax.ShapeDtypeStruct((NPAD, H), jnp.float32),
            jax.ShapeDtypeStruct((NPAD, 1), jnp.float32),
        ),
    )(xp, W1, degp)


def _tc2(aggp, g1, dis, W2, b1row):
    def body(aggp_ref, g1_ref, dis_ref, w_ref, b_ref, g2_ref):
        dis = dis_ref[...]
        h = jnp.maximum(
            (aggp_ref[0] + aggp_ref[1] + g1_ref[...]) * dis + b_ref[...], 0.0)
        t = jnp.dot(h, w_ref[...], preferred_element_type=jnp.float32)
        g2_ref[...] = t * dis

    return pl.pallas_call(
        body,
        out_shape=jax.ShapeDtypeStruct((NPAD, H), jnp.float32),
    )(aggp, g1, dis, W2, b1row)


def _tc3(aggp, g2, dis, b2row, brow, M1, mb1row, M2p, mb2row):
    def body(aggp_ref, g2_ref, dis_ref, b2_ref, brow_ref, m1_ref, mb1_ref,
             m2_ref, mb2_ref, out_ref):
        dis = dis_ref[...]
        h2 = jnp.maximum(
            (aggp_ref[0] + aggp_ref[1] + g2_ref[...]) * dis + b2_ref[...], 0.0)
        gids = lax.broadcasted_iota(jnp.int32, (B, NPAD), 0)
        oh = (brow_ref[...] == gids).astype(jnp.float32)
        ssum = jnp.dot(oh, h2, preferred_element_type=jnp.float32)
        cnt = jnp.dot(oh, jnp.ones((NPAD, 1), jnp.float32),
                      preferred_element_type=jnp.float32)
        pooled = ssum / jnp.maximum(cnt, 1.0)
        z = jnp.maximum(
            jnp.dot(pooled, m1_ref[...], preferred_element_type=jnp.float32)
            + mb1_ref[...], 0.0)
        out_ref[...] = (
            jnp.dot(z, m2_ref[...], preferred_element_type=jnp.float32)
            + mb2_ref[...])

    return pl.pallas_call(
        body,
        out_shape=jax.ShapeDtypeStruct((B, 128), jnp.float32),
    )(aggp, g2, dis, b2row, brow, M1, mb1row, M2p, mb2row)


@jax.jit
def kernel(x, ei, ew, b, W1, b1, W2, b2, M1, mb1, M2, mb2):
    xp = jnp.pad(x, ((0, NPAD - N), (0, 0)))
    npad_e = EPAD - E
    pidx = jnp.arange(npad_e, dtype=jnp.int32) % N
    src = jnp.concatenate([ei[0], pidx]).reshape(NW, C1, K1)
    dst = jnp.concatenate([ei[1], pidx]).reshape(NW, C1, K1)
    ewi = lax.bitcast_convert_type(
        jnp.concatenate([ew, jnp.zeros((npad_e,), jnp.float32)]),
        jnp.int32).reshape(NW, C1, K1)
    meta = jnp.stack([src, dst, ewi], axis=2)
    brow = jnp.pad(b, (0, NPAD - N), constant_values=B)[None, :]
    b1row = b1[None, :]
    b2row = b2[None, :]
    mb1row = mb1[None, :]
    M2p = jnp.pad(M2, ((0, 0), (0, 128 - M2.shape[1])))
    mb2row = jnp.pad(mb2, (0, 128 - mb2.shape[0]))[None, :]

    degp = _deg_kernel(meta)
    degp3 = degp[:, :, None]

    g1, dis = _tc1(xp, W1, degp3)
    agg1 = _agg_kernel(g1, meta)
    g2 = _tc2(agg1, g1, dis, W2, b1row)
    agg2 = _agg_kernel(g2, meta)
    out_full = _tc3(agg2, g2, dis, b2row, brow, M1, mb1row, M2p, mb2row)
    return out_full[:, :1]


# speedup vs baseline: 17.4868x; 1.1220x over previous
import functools

import jax
import jax.numpy as jnp
from jax import lax
from jax.experimental import pallas as pl
from jax.experimental.pallas import tpu as pltpu
from jax.experimental.pallas import tpu_sc as plsc

N = 10000
E = 320000
D = 128
H = 64
B = 32

NPAD = 10240
NC = 2
NS = 16
NW = NC * NS
K1 = 128
C1 = 81
EPW = C1 * K1
EPAD = NW * EPW
NBUF = 3
RPT = NPAD // NS
FG = H // 16


def _sc_mesh():
    return plsc.VectorSubcoreMesh(core_axis_name="c", subcore_axis_name="s")


_SC_PARAMS = pltpu.CompilerParams(use_tc_tiling_on_sc=False,
                                  needs_layout_passes=False)


def _deg_kernel(meta):
    @functools.partial(
        pl.kernel,
        out_type=jax.ShapeDtypeStruct((NC, NPAD), jnp.float32),
        mesh=_sc_mesh(),
        compiler_params=_SC_PARAMS,
        scratch_types=(
            [pltpu.VMEM((3, K1), jnp.int32)] * NBUF
            + [pltpu.VMEM((K1,), jnp.float32)] * NBUF
            + [pltpu.VMEM((RPT,), jnp.float32),
               pltpu.VMEM_SHARED((NPAD,), jnp.float32)]
            + [pltpu.SemaphoreType.DMA] * (2 * NBUF)
        ),
    )
    def k(meta_h, out_h, ed0, ed1, ed2, ef0, ef1, ef2, zbuf, acc,
          sm0, sm1, sm2, ss0, ss1, ss2):
        ed = [ed0, ed1, ed2]
        ef = [ef0, ef1, ef2]
        sm = [sm0, sm1, sm2]
        ss = [ss0, ss1, ss2]
        cid = lax.axis_index("c")
        sid = lax.axis_index("s")
        wid = cid * NS + sid

        for i in range(RPT // 16):
            zbuf[pl.ds(i * 16, 16)] = jnp.zeros((16,), jnp.float32)
        pltpu.sync_copy(zbuf, acc.at[pl.ds(sid * RPT, RPT)])
        plsc.subcore_barrier()

        pltpu.async_copy(meta_h.at[wid, 0], ed[0], sm[0])
        pltpu.async_copy(meta_h.at[wid, 1], ed[1], sm[1])

        def body(gi, carry):
            sdescs = [None] * NBUF
            for b in range(NBUF):
                c = gi * NBUF + b
                s, s2 = b, (b + 2) % NBUF
                pltpu.make_async_copy(meta_h.at[wid, c], ed[s], sm[s]).wait()
                for gg in range(K1 // 16):
                    sl = pl.ds(gg * 16, 16)
                    ef[s][sl] = plsc.bitcast(ed[s][2, sl], jnp.float32)

                sdescs[b] = pltpu.async_copy(ef[s], acc.at[ed[s].at[1]],
                                             ss[s], add=True)
                if b >= 1:
                    sdescs[b - 1].wait()

                @pl.when(c + 2 < C1)
                def _():
                    pltpu.async_copy(meta_h.at[wid, c + 2], ed[s2], sm[s2])
            sdescs[NBUF - 1].wait()
            return carry

        lax.fori_loop(0, C1 // NBUF, body, 0)
        plsc.subcore_barrier()
        pltpu.sync_copy(acc.at[pl.ds(sid * RPT, RPT)],
                        out_h.at[cid, pl.ds(sid * RPT, RPT)])

    return k(meta)


def _agg_kernel(g, meta):
    @functools.partial(
        pl.kernel,
        out_type=jax.ShapeDtypeStruct((NC, NPAD, H), jnp.float32),
        mesh=_sc_mesh(),
        compiler_params=_SC_PARAMS,
        scratch_types=(
            [pltpu.VMEM((3, K1), jnp.int32)] * NBUF
            + [pltpu.VMEM((K1, H), jnp.float32)] * NBUF
            + [pltpu.VMEM((64, H), jnp.float32),
               pltpu.VMEM_SHARED((NPAD, H), jnp.float32)]
            + [pltpu.SemaphoreType.DMA] * (3 * NBUF)
        ),
    )
    def k(g_h, meta_h, out_h, ed0, ed1, ed2, r0, r1, r2, zbuf, acc,
          sm0, sm1, sm2, sg0, sg1, sg2, ss0, ss1, ss2):
        ed = [ed0, ed1, ed2]
        rows = [r0, r1, r2]
        sm = [sm0, sm1, sm2]
        sg = [sg0, sg1, sg2]
        ss = [ss0, ss1, ss2]
        cid = lax.axis_index("c")
        sid = lax.axis_index("s")
        wid = cid * NS + sid

        def zfill(i, carry):
            for j in range(FG):
                zbuf[i, pl.ds(j * 16, 16)] = jnp.zeros((16,), jnp.float32)
            return carry

        lax.fori_loop(0, 64, zfill, 0)

        def zcopy(i, carry):
            pltpu.sync_copy(zbuf, acc.at[pl.ds(sid * RPT + i * 64, 64)])
            return carry

        lax.fori_loop(0, RPT // 64, zcopy, 0)
        plsc.subcore_barrier()

        pltpu.async_copy(meta_h.at[wid, 0], ed[0], sm[0])
        pltpu.async_copy(meta_h.at[wid, 1], ed[1], sm[1])
        pltpu.make_async_copy(meta_h.at[wid, 0], ed[0], sm[0]).wait()
        pltpu.async_copy(g_h.at[ed[0].at[0]], rows[0], sg[0])

        def body(gi, carry):
            sdescs = [None] * NBUF
            for b in range(NBUF):
                c = gi * NBUF + b
                s, s1, s2 = b, (b + 1) % NBUF, (b + 2) % NBUF
                pltpu.make_async_copy(
                    g_h.at[ed[s].at[0]], rows[s], sg[s]).wait()

                @pl.when(c + 1 < C1)
                def _():
                    pltpu.make_async_copy(
                        meta_h.at[wid, c + 1], ed[s1], sm[s1]).wait()
                    pltpu.async_copy(g_h.at[ed[s1].at[0]], rows[s1], sg[s1])

                rs = rows[s]

                def scale(gg, c2):
                    ew16 = plsc.bitcast(ed[s][2, pl.ds(gg * 16, 16)],
                                        jnp.float32)
                    for j in range(16):
                        e = gg * 16 + j
                        w = ew16[j]
                        for f in range(FG):
                            sl = pl.ds(f * 16, 16)
                            rs[e, sl] = rs[e, sl] * w
                    return c2

                lax.fori_loop(0, K1 // 16, scale, 0)

                sdescs[b] = pltpu.async_copy(rows[s], acc.at[ed[s].at[1]],
                                             ss[s], add=True)
                if b >= 1:
                    sdescs[b - 1].wait()

                @pl.when(c + 2 < C1)
                def _():
                    pltpu.async_copy(meta_h.at[wid, c + 2], ed[s2], sm[s2])
            sdescs[NBUF - 1].wait()
            return carry

        lax.fori_loop(0, C1 // NBUF, body, 0)
        plsc.subcore_barrier()
        pltpu.sync_copy(acc.at[pl.ds(sid * RPT, RPT)],
                        out_h.at[cid, pl.ds(sid * RPT, RPT)])

    return k(g, meta)


def _tc1(xp, W1, degp):
    def body(x_ref, w_ref, degp_ref, g_ref, dis_ref):
        deg = degp_ref[0] + degp_ref[1] + 1.0
        dis = lax.rsqrt(deg)
        h = jnp.dot(x_ref[...], w_ref[...], preferred_element_type=jnp.float32)
        g_ref[...] = h * dis
        dis_ref[...] = dis

    return pl.pallas_call(
        body,
        out_shape=(
            jax.ShapeDtypeStruct((NPAD, H), jnp.float32),
            jax.ShapeDtypeStruct((NPAD, 1), jnp.float32),
        ),
    )(xp, W1, degp)


def _tc2(aggp, g1, dis, W2, b1row):
    def body(aggp_ref, g1_ref, dis_ref, w_ref, b_ref, g2_ref):
        dis = dis_ref[...]
        h = jnp.maximum(
            (aggp_ref[0] + aggp_ref[1] + g1_ref[...]) * dis + b_ref[...], 0.0)
        t = jnp.dot(h, w_ref[...], preferred_element_type=jnp.float32)
        g2_ref[...] = t * dis

    return pl.pallas_call(
        body,
        out_shape=jax.ShapeDtypeStruct((NPAD, H), jnp.float32),
    )(aggp, g1, dis, W2, b1row)


def _tc3(aggp, g2, dis, b2row, brow, M1, mb1row, M2p, mb2row):
    def body(aggp_ref, g2_ref, dis_ref, b2_ref, brow_ref, m1_ref, mb1_ref,
             m2_ref, mb2_ref, out_ref):
        dis = dis_ref[...]
        h2 = jnp.maximum(
            (aggp_ref[0] + aggp_ref[1] + g2_ref[...]) * dis + b2_ref[...], 0.0)
        gids = lax.broadcasted_iota(jnp.int32, (B, NPAD), 0)
        oh = (brow_ref[...] == gids).astype(jnp.float32)
        ssum = jnp.dot(oh, h2, preferred_element_type=jnp.float32)
        cnt = jnp.dot(oh, jnp.ones((NPAD, 1), jnp.float32),
                      preferred_element_type=jnp.float32)
        pooled = ssum / jnp.maximum(cnt, 1.0)
        z = jnp.maximum(
            jnp.dot(pooled, m1_ref[...], preferred_element_type=jnp.float32)
            + mb1_ref[...], 0.0)
        out_ref[...] = (
            jnp.dot(z, m2_ref[...], preferred_element_type=jnp.float32)
            + mb2_ref[...])

    return pl.pallas_call(
        body,
        out_shape=jax.ShapeDtypeStruct((B, 128), jnp.float32),
    )(aggp, g2, dis, b2row, brow, M1, mb1row, M2p, mb2row)


@jax.jit
def kernel(x, ei, ew, b, W1, b1, W2, b2, M1, mb1, M2, mb2):
    xp = jnp.pad(x, ((0, NPAD - N), (0, 0)))
    npad_e = EPAD - E
    pidx = jnp.arange(npad_e, dtype=jnp.int32) % N
    src = jnp.concatenate([ei[0], pidx]).reshape(NW, C1, K1)
    dst = jnp.concatenate([ei[1], pidx]).reshape(NW, C1, K1)
    ewi = lax.bitcast_convert_type(
        jnp.concatenate([ew, jnp.zeros((npad_e,), jnp.float32)]),
        jnp.int32).reshape(NW, C1, K1)
    meta = jnp.stack([src, dst, ewi], axis=2)
    brow = jnp.pad(b, (0, NPAD - N), constant_values=B)[None, :]
    b1row = b1[None, :]
    b2row = b2[None, :]
    mb1row = mb1[None, :]
    M2p = jnp.pad(M2, ((0, 0), (0, 128 - M2.shape[1])))
    mb2row = jnp.pad(mb2, (0, 128 - mb2.shape[0]))[None, :]

    degp = _deg_kernel(meta)
    degp3 = degp[:, :, None]

    g1, dis = _tc1(xp, W1, degp3)
    agg1 = _agg_kernel(g1, meta)
    g2 = _tc2(agg1, g1, dis, W2, b1row)
    agg2 = _agg_kernel(g2, meta)
    out_full = _tc3(agg2, g2, dis, b2row, brow, M1, mb1row, M2p, mb2row)
    return out_full[:, :1]


# speedup vs baseline: 20.7061x; 1.1841x over previous
import functools

import jax
import jax.numpy as jnp
from jax import lax
from jax.experimental import pallas as pl
from jax.experimental.pallas import tpu as pltpu
from jax.experimental.pallas import tpu_sc as plsc

N = 10000
E = 320000
D = 128
H = 64
B = 32

NPAD = 10240
NC = 2
NS = 16
NW = NC * NS
K1 = 128
C1 = 81
EPW = C1 * K1
EPAD = NW * EPW
NBUF = 3
RPT = NPAD // NS
FG = H // 16

PERM = []
for _g in range(H // 32):
    for _k in range(16):
        PERM.append(32 * _g + _k)
        PERM.append(32 * _g + 16 + _k)


def _sc_mesh():
    return plsc.VectorSubcoreMesh(core_axis_name="c", subcore_axis_name="s")


_SC_PARAMS = pltpu.CompilerParams(use_tc_tiling_on_sc=False,
                                  needs_layout_passes=False)


def _deg_kernel(meta):
    @functools.partial(
        pl.kernel,
        out_type=jax.ShapeDtypeStruct((NC, NPAD), jnp.float32),
        mesh=_sc_mesh(),
        compiler_params=_SC_PARAMS,
        scratch_types=(
            [pltpu.VMEM((3, K1), jnp.int32)] * NBUF
            + [pltpu.VMEM((K1,), jnp.float32)] * NBUF
            + [pltpu.VMEM((RPT,), jnp.float32),
               pltpu.VMEM_SHARED((NPAD,), jnp.float32)]
            + [pltpu.SemaphoreType.DMA] * (2 * NBUF)
        ),
    )
    def k(meta_h, out_h, ed0, ed1, ed2, ef0, ef1, ef2, zbuf, acc,
          sm0, sm1, sm2, ss0, ss1, ss2):
        ed = [ed0, ed1, ed2]
        ef = [ef0, ef1, ef2]
        sm = [sm0, sm1, sm2]
        ss = [ss0, ss1, ss2]
        cid = lax.axis_index("c")
        sid = lax.axis_index("s")
        wid = cid * NS + sid

        for i in range(RPT // 16):
            zbuf[pl.ds(i * 16, 16)] = jnp.zeros((16,), jnp.float32)
        pltpu.sync_copy(zbuf, acc.at[pl.ds(sid * RPT, RPT)])
        plsc.subcore_barrier()

        pltpu.async_copy(meta_h.at[wid, 0], ed[0], sm[0])
        pltpu.async_copy(meta_h.at[wid, 1], ed[1], sm[1])

        def body(gi, carry):
            sdescs = [None] * NBUF
            for b in range(NBUF):
                c = gi * NBUF + b
                s, s2 = b, (b + 2) % NBUF
                pltpu.make_async_copy(meta_h.at[wid, c], ed[s], sm[s]).wait()
                for gg in range(K1 // 16):
                    sl = pl.ds(gg * 16, 16)
                    ef[s][sl] = plsc.bitcast(ed[s][2, sl], jnp.float32)

                sdescs[b] = pltpu.async_copy(ef[s], acc.at[ed[s].at[1]],
                                             ss[s], add=True)
                if b >= 1:
                    sdescs[b - 1].wait()

                @pl.when(c + 2 < C1)
                def _():
                    pltpu.async_copy(meta_h.at[wid, c + 2], ed[s2], sm[s2])
            sdescs[NBUF - 1].wait()
            return carry

        lax.fori_loop(0, C1 // NBUF, body, 0)
        plsc.subcore_barrier()
        pltpu.sync_copy(acc.at[pl.ds(sid * RPT, RPT)],
                        out_h.at[cid, pl.ds(sid * RPT, RPT)])

    return k(meta)


def _agg_kernel(g, meta):
    @functools.partial(
        pl.kernel,
        out_type=jax.ShapeDtypeStruct((NC, NPAD, H), jnp.float32),
        mesh=_sc_mesh(),
        compiler_params=_SC_PARAMS,
        scratch_types=(
            [pltpu.VMEM((3, K1), jnp.int32)] * NBUF
            + [pltpu.VMEM((K1, H), jnp.bfloat16)] * NBUF
            + [pltpu.VMEM((K1, H), jnp.float32)] * NBUF
            + [pltpu.VMEM((64, H), jnp.float32),
               pltpu.VMEM_SHARED((NPAD, H), jnp.float32)]
            + [pltpu.SemaphoreType.DMA] * (3 * NBUF)
        ),
    )
    def k(g_h, meta_h, out_h, ed0, ed1, ed2, r0, r1, r2, f0, f1, f2,
          zbuf, acc, sm0, sm1, sm2, sg0, sg1, sg2, ss0, ss1, ss2):
        ed = [ed0, ed1, ed2]
        rows = [r0, r1, r2]
        fb = [f0, f1, f2]
        sm = [sm0, sm1, sm2]
        sg = [sg0, sg1, sg2]
        ss = [ss0, ss1, ss2]
        cid = lax.axis_index("c")
        sid = lax.axis_index("s")
        wid = cid * NS + sid

        def zfill(i, carry):
            for j in range(FG):
                zbuf[i, pl.ds(j * 16, 16)] = jnp.zeros((16,), jnp.float32)
            return carry

        lax.fori_loop(0, 64, zfill, 0)

        def zcopy(i, carry):
            pltpu.sync_copy(zbuf, acc.at[pl.ds(sid * RPT + i * 64, 64)])
            return carry

        lax.fori_loop(0, RPT // 64, zcopy, 0)
        plsc.subcore_barrier()

        pltpu.async_copy(meta_h.at[wid, 0], ed[0], sm[0])
        pltpu.async_copy(meta_h.at[wid, 1], ed[1], sm[1])
        pltpu.make_async_copy(meta_h.at[wid, 0], ed[0], sm[0]).wait()
        pltpu.async_copy(g_h.at[ed[0].at[0]], rows[0], sg[0])

        def body(gi, carry):
            sdescs = [None] * NBUF
            for b in range(NBUF):
                c = gi * NBUF + b
                s, s1, s2 = b, (b + 1) % NBUF, (b + 2) % NBUF
                pltpu.make_async_copy(
                    g_h.at[ed[s].at[0]], rows[s], sg[s]).wait()

                @pl.when(c + 1 < C1)
                def _():
                    pltpu.make_async_copy(
                        meta_h.at[wid, c + 1], ed[s1], sm[s1]).wait()
                    pltpu.async_copy(g_h.at[ed[s1].at[0]], rows[s1], sg[s1])

                rs = rows[s]
                fs = fb[s]

                def scale(gg, c2):
                    ew16 = plsc.bitcast(ed[s][2, pl.ds(gg * 16, 16)],
                                        jnp.float32)
                    for j in range(16):
                        e = gg * 16 + j
                        w = ew16[j]
                        for f in range(H // 32):
                            x32 = rs[e, pl.ds(f * 32, 32)]
                            lo, hi = plsc.unpack(
                                x32, format=plsc.PackFormat.INTERLEAVED)
                            fs[e, pl.ds(f * 32, 16)] = lo * w
                            fs[e, pl.ds(f * 32 + 16, 16)] = hi * w
                    return c2

                lax.fori_loop(0, K1 // 16, scale, 0)

                sdescs[b] = pltpu.async_copy(fb[s], acc.at[ed[s].at[1]],
                                             ss[s], add=True)
                if b >= 1:
                    sdescs[b - 1].wait()

                @pl.when(c + 2 < C1)
                def _():
                    pltpu.async_copy(meta_h.at[wid, c + 2], ed[s2], sm[s2])
            sdescs[NBUF - 1].wait()
            return carry

        lax.fori_loop(0, C1 // NBUF, body, 0)
        plsc.subcore_barrier()
        pltpu.sync_copy(acc.at[pl.ds(sid * RPT, RPT)],
                        out_h.at[cid, pl.ds(sid * RPT, RPT)])

    return k(g, meta)


def _tc1(xp, W1, W1p, degp):
    def body(x_ref, w_ref, wp_ref, degp_ref, g_ref, gb_ref, dis_ref):
        deg = degp_ref[0] + degp_ref[1] + 1.0
        dis = lax.rsqrt(deg)
        x = x_ref[...]
        h = jnp.dot(x, w_ref[...], preferred_element_type=jnp.float32)
        hp = jnp.dot(x, wp_ref[...], preferred_element_type=jnp.float32)
        g_ref[...] = h * dis
        gb_ref[...] = (hp * dis).astype(jnp.bfloat16)
        dis_ref[...] = dis

    return pl.pallas_call(
        body,
        out_shape=(
            jax.ShapeDtypeStruct((NPAD, H), jnp.float32),
            jax.ShapeDtypeStruct((NPAD, H), jnp.bfloat16),
            jax.ShapeDtypeStruct((NPAD, 1), jnp.float32),
        ),
    )(xp, W1, W1p, degp)


def _tc2(aggp, g1, dis, W2, W2p, b1row):
    def body(aggp_ref, g1_ref, dis_ref, w_ref, wp_ref, b_ref, g2_ref, gb_ref):
        dis = dis_ref[...]
        h = jnp.maximum(
            (aggp_ref[0] + aggp_ref[1] + g1_ref[...]) * dis + b_ref[...], 0.0)
        t = jnp.dot(h, w_ref[...], preferred_element_type=jnp.float32)
        tp = jnp.dot(h, wp_ref[...], preferred_element_type=jnp.float32)
        g2_ref[...] = t * dis
        gb_ref[...] = (tp * dis).astype(jnp.bfloat16)

    return pl.pallas_call(
        body,
        out_shape=(
            jax.ShapeDtypeStruct((NPAD, H), jnp.float32),
            jax.ShapeDtypeStruct((NPAD, H), jnp.bfloat16),
        ),
    )(aggp, g1, dis, W2, W2p, b1row)


def _tc3(aggp, g2, dis, b2row, brow, M1, mb1row, M2p, mb2row):
    def body(aggp_ref, g2_ref, dis_ref, b2_ref, brow_ref, m1_ref, mb1_ref,
             m2_ref, mb2_ref, out_ref):
        dis = dis_ref[...]
        h2 = jnp.maximum(
            (aggp_ref[0] + aggp_ref[1] + g2_ref[...]) * dis + b2_ref[...], 0.0)
        gids = lax.broadcasted_iota(jnp.int32, (B, NPAD), 0)
        oh = (brow_ref[...] == gids).astype(jnp.float32)
        ssum = jnp.dot(oh, h2, preferred_element_type=jnp.float32)
        cnt = jnp.dot(oh, jnp.ones((NPAD, 1), jnp.float32),
                      preferred_element_type=jnp.float32)
        pooled = ssum / jnp.maximum(cnt, 1.0)
        z = jnp.maximum(
            jnp.dot(pooled, m1_ref[...], preferred_element_type=jnp.float32)
            + mb1_ref[...], 0.0)
        out_ref[...] = (
            jnp.dot(z, m2_ref[...], preferred_element_type=jnp.float32)
            + mb2_ref[...])

    return pl.pallas_call(
        body,
        out_shape=jax.ShapeDtypeStruct((B, 128), jnp.float32),
    )(aggp, g2, dis, b2row, brow, M1, mb1row, M2p, mb2row)


@jax.jit
def kernel(x, ei, ew, b, W1, b1, W2, b2, M1, mb1, M2, mb2):
    xp = jnp.pad(x, ((0, NPAD - N), (0, 0)))
    npad_e = EPAD - E
    pidx = jnp.arange(npad_e, dtype=jnp.int32) % N
    src = jnp.concatenate([ei[0], pidx]).reshape(NW, C1, K1)
    dst = jnp.concatenate([ei[1], pidx]).reshape(NW, C1, K1)
    ewi = lax.bitcast_convert_type(
        jnp.concatenate([ew, jnp.zeros((npad_e,), jnp.float32)]),
        jnp.int32).reshape(NW, C1, K1)
    meta = jnp.stack([src, dst, ewi], axis=2)
    brow = jnp.pad(b, (0, NPAD - N), constant_values=B)[None, :]
    b1row = b1[None, :]
    b2row = b2[None, :]
    mb1row = mb1[None, :]
    M2p = jnp.pad(M2, ((0, 0), (0, 128 - M2.shape[1])))
    mb2row = jnp.pad(mb2, (0, 128 - mb2.shape[0]))[None, :]

    perm = jnp.asarray(PERM, dtype=jnp.int32)
    W1p = jnp.take(W1, perm, axis=1)
    W2p = jnp.take(W2, perm, axis=1)

    degp = _deg_kernel(meta)
    degp3 = degp[:, :, None]

    g1, gb1, dis = _tc1(xp, W1, W1p, degp3)
    agg1 = _agg_kernel(gb1, meta)
    g2, gb2 = _tc2(agg1, g1, dis, W2, W2p, b1row)
    agg2 = _agg_kernel(gb2, meta)
    out_full = _tc3(agg2, g2, dis, b2row, brow, M1, mb1row, M2p, mb2row)
    return out_full[:, :1]
